# baseline (speedup 1.0000x reference)
import functools

import jax
import jax.numpy as jnp
from jax import lax
from jax.experimental import pallas as pl
from jax.experimental.pallas import tpu as pltpu
from jax.experimental.pallas import tpu_sc as plsc

K_NEURONS = 10000
FEAT = 784
BATCH = 4096

BB = 512
BK = 2000
NB = BATCH // BB
NK = K_NEURONS // BK

BIG = 3.0e38

FEAT_PAD = 896

SC_NC = 2
SC_NS = 16
SC_NW = SC_NC * SC_NS
ROWS_PER_W = BATCH // 2 // SC_NW


def _dist_argmin_body(x_ref, w_ref, idx_ref, wsq_ref):
    i = pl.program_id(0)
    x = x_ref[...]
    x2 = x * -2.0
    xsq = jnp.sum(x * x, axis=1, keepdims=True)

    @pl.when(i == 0)
    def _wsq():
        w = w_ref[...]
        wsq_ref[...] = jnp.sum(w * w, axis=1, keepdims=True).reshape(1, K_NEURONS)

    best = jnp.full((BB, 1), BIG, jnp.float32)
    bidx = jnp.zeros((BB, 1), jnp.int32)
    for j in range(NK):
        wblk = w_ref[pl.ds(j * BK, BK), :]
        cross2 = lax.dot_general(x2, wblk, (((1,), (1,)), ((), ())),
                                 preferred_element_type=jnp.float32)
        dist = (xsq + cross2) + wsq_ref[:, pl.ds(j * BK, BK)]
        m = jnp.min(dist, axis=1, keepdims=True)
        am = jnp.argmin(dist, axis=1).astype(jnp.int32)[:, None] + j * BK
        upd = m < best
        best = jnp.where(upd, m, best)
        bidx = jnp.where(upd, am, bidx)
    idx_ref[...] = bidx.reshape(BB)


HALF = BATCH // 2
NB_H = HALF // BB


def _bmu_indices(x, weights, chunk):
    return pl.pallas_call(
        _dist_argmin_body,
        grid=(NB_H,),
        in_specs=[
            pl.BlockSpec((BB, FEAT), lambda i: (chunk * NB_H + i, 0)),
            pl.BlockSpec((K_NEURONS, FEAT), lambda i: (0, 0)),
        ],
        out_specs=pl.BlockSpec((BB,), lambda i: (i,)),
        out_shape=jax.ShapeDtypeStruct((HALF,), jnp.int32),
        scratch_shapes=[
            pltpu.VMEM((1, K_NEURONS), jnp.float32),
        ],
        compiler_params=pltpu.CompilerParams(
            dimension_semantics=("arbitrary",),
            vmem_limit_bytes=100 * 1024 * 1024),
    )(x, weights)


def _gather_body(table_hbm, idx_hbm, out_hbm, idx_v, rows_v, sem):
    wid = lax.axis_index("s") * SC_NC + lax.axis_index("c")
    base = wid * ROWS_PER_W
    pltpu.sync_copy(idx_hbm.at[pl.ds(base, ROWS_PER_W)], idx_v)
    pltpu.async_copy(table_hbm.at[idx_v], rows_v, sem).wait()
    pltpu.sync_copy(rows_v, out_hbm.at[pl.ds(base, ROWS_PER_W)])


def _gather_rows(table, idx):
    mesh = plsc.VectorSubcoreMesh(core_axis_name="c", subcore_axis_name="s")
    return pl.kernel(
        _gather_body,
        out_type=jax.ShapeDtypeStruct((BATCH // 2, FEAT_PAD), jnp.float32),
        mesh=mesh,
        scratch_types=[
            pltpu.VMEM((ROWS_PER_W,), jnp.int32),
            pltpu.VMEM((ROWS_PER_W, FEAT_PAD), jnp.float32),
            pltpu.SemaphoreType.DMA,
        ],
    )(table, idx)


def kernel(inputs, weights):
    x = inputs.reshape(-1, FEAT)
    table = jnp.pad(weights, ((0, 0), (0, FEAT_PAD - FEAT)))
    idx_a = _bmu_indices(x, weights, 0)
    bmu_a = _gather_rows(table, idx_a)
    idx_b = _bmu_indices(x, weights, 1)
    bmu_b = _gather_rows(table, idx_b)
    return jnp.concatenate([bmu_a, bmu_b], axis=0)[:, :FEAT]


# speedup vs baseline: 1.1871x; 1.1871x over previous
import functools

import jax
import jax.numpy as jnp
from jax import lax
from jax.experimental import pallas as pl
from jax.experimental.pallas import tpu as pltpu
from jax.experimental.pallas import tpu_sc as plsc

K_NEURONS = 10000
FEAT = 784
BATCH = 4096

BB = 1024
BK = 2000
NB = BATCH // BB
NK = K_NEURONS // BK

BIG = 3.0e38

FEAT_PAD = 896

SC_NC = 2
SC_NS = 16
SC_NW = SC_NC * SC_NS
ROWS_PER_W = BATCH // SC_NW


def _dist_argmin_body(x_ref, w_ref, idx_ref, wsq_ref):
    i = pl.program_id(0)
    x = x_ref[...]
    x2 = x * -2.0
    xsq = jnp.sum(x * x, axis=1, keepdims=True)

    @pl.when(i == 0)
    def _wsq():
        w = w_ref[...]
        wsq_ref[...] = jnp.sum(w * w, axis=1, keepdims=True).reshape(1, K_NEURONS)

    best = jnp.full((BB, 1), BIG, jnp.float32)
    bidx = jnp.zeros((BB, 1), jnp.int32)
    for j in range(NK):
        wblk = w_ref[pl.ds(j * BK, BK), :]
        cross2 = lax.dot_general(x2, wblk, (((1,), (1,)), ((), ())),
                                 preferred_element_type=jnp.float32)
        dist = (xsq + cross2) + wsq_ref[:, pl.ds(j * BK, BK)]
        m = jnp.min(dist, axis=1, keepdims=True)
        am = jnp.argmin(dist, axis=1).astype(jnp.int32)[:, None] + j * BK
        upd = m < best
        best = jnp.where(upd, m, best)
        bidx = jnp.where(upd, am, bidx)
    idx_ref[...] = bidx.reshape(BB)


def _bmu_indices(x, weights):
    return pl.pallas_call(
        _dist_argmin_body,
        grid=(NB,),
        in_specs=[
            pl.BlockSpec((BB, FEAT), lambda i: (i, 0)),
            pl.BlockSpec((K_NEURONS, FEAT), lambda i: (0, 0)),
        ],
        out_specs=pl.BlockSpec((BB,), lambda i: (i,)),
        out_shape=jax.ShapeDtypeStruct((BATCH,), jnp.int32),
        scratch_shapes=[
            pltpu.VMEM((1, K_NEURONS), jnp.float32),
        ],
        compiler_params=pltpu.CompilerParams(
            dimension_semantics=("arbitrary",),
            vmem_limit_bytes=100 * 1024 * 1024),
    )(x, weights)


def _gather_body(table_hbm, idx_hbm, out_hbm, idx_v, rows_v, sem):
    wid = lax.axis_index("s") * SC_NC + lax.axis_index("c")
    base = wid * ROWS_PER_W
    pltpu.sync_copy(idx_hbm.at[pl.ds(base, ROWS_PER_W)], idx_v)
    pltpu.async_copy(table_hbm.at[idx_v], rows_v, sem).wait()
    pltpu.sync_copy(rows_v, out_hbm.at[pl.ds(base, ROWS_PER_W)])


def _gather_rows(table, idx):
    mesh = plsc.VectorSubcoreMesh(core_axis_name="c", subcore_axis_name="s")
    return pl.kernel(
        _gather_body,
        out_type=jax.ShapeDtypeStruct((BATCH, FEAT_PAD), jnp.float32),
        mesh=mesh,
        scratch_types=[
            pltpu.VMEM((ROWS_PER_W,), jnp.int32),
            pltpu.VMEM((ROWS_PER_W, FEAT_PAD), jnp.float32),
            pltpu.SemaphoreType.DMA,
        ],
    )(table, idx)


def kernel(inputs, weights):
    x = inputs.reshape(-1, FEAT)
    idx = _bmu_indices(x, weights)
    table = jnp.pad(weights, ((0, 0), (0, FEAT_PAD - FEAT)))
    return _gather_rows(table, idx)[:, :FEAT]
